# baseline (speedup 1.0000x reference)
import functools
import jax
import jax.numpy as jnp
from jax import lax
from jax.experimental import pallas as pl
from jax.experimental.pallas import tpu as pltpu
from jax.experimental.pallas import tpu_sc as plsc

N = 100000
E = 6400000
IN_DIM = 4
HID = 32

NC = 2
NS = 16

L1_LANES, L1_SUB = 1024, 3
L2_LANES, L2_SUB = 384, 2
C1 = L1_LANES * L1_SUB
C2 = L2_LANES * L2_SUB

import math
_GRAIN = math.lcm(NC * NS * C1, NS * C2)
EP = (E + _GRAIN - 1) // _GRAIN * _GRAIN
EPX = EP + max(C1, C2)
NACC = ((N + 1) + NS * 8 - 1) // (NS * 8) * (NS * 8)
ZCH = NACC // NS

_MESH = plsc.VectorSubcoreMesh(core_axis_name="c", subcore_axis_name="s")


def _sc_agg(table, idx2d, dst2d, zeros, split_edges, lanes, sub, width):
    chunk = lanes * sub
    idxrows = EP // lanes
    rows_per_sub = idxrows // ((NC * NS) if split_edges else NS)
    n_bodies = rows_per_sub // sub
    stacked = table.shape[0] == NC

    @functools.partial(
        pl.kernel,
        out_type=jax.ShapeDtypeStruct((NC, NACC, width), jnp.float32),
        mesh=_MESH,
        scratch_types=[
            pltpu.VMEM((2, sub, lanes), jnp.int32),
            pltpu.VMEM((2, sub, lanes), jnp.int32),
            pltpu.VMEM((2 * chunk, width), jnp.float32),
            pltpu.VMEM_SHARED((NACC, width), jnp.float32),
            pltpu.SemaphoreType.DMA,
            pltpu.SemaphoreType.DMA,
            pltpu.SemaphoreType.DMA,
            pltpu.SemaphoreType.DMA,
        ],
        compiler_params=pltpu.CompilerParams(use_tc_tiling_on_sc=False),
    )
    def k(table_h, idx_h, dst_h, zeros_h, out_h,
          sbuf, dbuf, rows_v, acc, gsem, ssem, isem, dsem):
        c = lax.axis_index("c")
        s = lax.axis_index("s")
        tab = table_h.at[c] if stacked else table_h.at[0]
        zb = s * ZCH
        pltpu.sync_copy(zeros_h.at[pl.ds(zb, ZCH)], acc.at[pl.ds(zb, ZCH)])
        plsc.subcore_barrier()

        if split_edges:
            row0 = (c * NS + s) * rows_per_sub
        else:
            row0 = s * rows_per_sub

        def drain_scatters(sem):
            for j in range(sub):
                pltpu.make_async_copy(
                    zeros_h.at[pl.ds(0, lanes)],
                    rows_v.at[pl.ds(j * lanes, lanes)], sem).wait()

        def drain_ibuf(buf, sem):
            pltpu.make_async_copy(idx_h.at[pl.ds(row0, sub)], buf, sem).wait()

        pltpu.sync_copy(idx_h.at[pl.ds(row0, sub)], sbuf.at[0])

        def body(i, carry):
            p = i & 1

            @pl.when(i > 1)
            def _():
                drain_scatters(ssem)

            pltpu.async_copy(dst_h.at[pl.ds(row0 + i * sub, sub)],
                             dbuf.at[p], dsem)

            @pl.when(i > 0)
            def _():
                drain_ibuf(sbuf.at[p], isem)

            g = [
                pltpu.async_copy(tab.at[sbuf.at[p, j]],
                                 rows_v.at[pl.ds(p * chunk + j * lanes, lanes)],
                                 gsem)
                for j in range(sub)
            ]
            pltpu.async_copy(idx_h.at[pl.ds(row0 + (i + 1) * sub, sub)],
                             sbuf.at[1 - p], isem)
            for cp in g:
                cp.wait()
            drain_ibuf(dbuf.at[p], dsem)
            for j in range(sub):
                pltpu.async_copy(rows_v.at[pl.ds(p * chunk + j * lanes, lanes)],
                                 acc.at[dbuf.at[p, j]], ssem, add=True)
            return carry

        lax.fori_loop(0, n_bodies, body, 0)
        drain_scatters(ssem)
        drain_scatters(ssem)
        drain_ibuf(sbuf.at[0], isem)
        plsc.subcore_barrier()
        pltpu.sync_copy(acc.at[pl.ds(zb, ZCH)], out_h.at[c, pl.ds(zb, ZCH)])

    return k(table, idx2d, dst2d, zeros)


BR = 1000


def _tc_layer1(P, W1p, b1):
    def body(p_ref, w_ref, b_ref, h_ref, dinv_ref):
        sblk = p_ref[0] + p_ref[1]
        dinv = 1.0 / jnp.maximum(sblk[:, 4:5], 1.0)
        mean = sblk * dinv
        h = jnp.dot(mean, w_ref[...], preferred_element_type=jnp.float32)
        h = jnp.maximum(h + b_ref[...], 0.0)
        h_ref[0] = h[:, :16]
        h_ref[1] = h[:, 16:]
        dinv_ref[...] = jnp.broadcast_to(dinv, (BR, 8))

    return pl.pallas_call(
        body,
        grid=(N // BR,),
        in_specs=[
            pl.BlockSpec((NC, BR, 8), lambda i: (0, i, 0)),
            pl.BlockSpec((8, HID), lambda i: (0, 0)),
            pl.BlockSpec((1, HID), lambda i: (0, 0)),
        ],
        out_specs=[
            pl.BlockSpec((NC, BR, 16), lambda i: (0, i, 0)),
            pl.BlockSpec((BR, 8), lambda i: (i, 0)),
        ],
        out_shape=[
            jax.ShapeDtypeStruct((NC, N, 16), jnp.float32),
            jax.ShapeDtypeStruct((N, 8), jnp.float32),
        ],
    )(P, W1p, b1)


def _tc_layer2(A, dinv8, W2s, b2):
    def body(a_ref, d_ref, w_ref, b_ref, o_ref):
        acc = jnp.dot(a_ref[0], w_ref[0], preferred_element_type=jnp.float32)
        acc += jnp.dot(a_ref[1], w_ref[1], preferred_element_type=jnp.float32)
        o_ref[...] = jnp.maximum(acc * d_ref[:, 0:1] + b_ref[...], 0.0)

    return pl.pallas_call(
        body,
        grid=(N // BR,),
        in_specs=[
            pl.BlockSpec((NC, BR, 16), lambda i: (0, i, 0)),
            pl.BlockSpec((BR, 8), lambda i: (i, 0)),
            pl.BlockSpec((NC, 16, HID), lambda i: (0, 0, 0)),
            pl.BlockSpec((1, HID), lambda i: (0, 0)),
        ],
        out_specs=pl.BlockSpec((BR, HID), lambda i: (i, 0)),
        out_shape=jax.ShapeDtypeStruct((N, HID), jnp.float32),
    )(A, dinv8, W2s, b2)


@jax.jit
def kernel(x, edge_index, W1, b1, W2, b2):
    src = edge_index[0]
    dst = edge_index[1]

    pad = EPX - E
    src_p = jnp.concatenate([src, jnp.zeros((pad,), jnp.int32)])
    dst_p = jnp.concatenate([dst, jnp.full((pad,), N, jnp.int32)])

    zeros8 = jnp.zeros((NACC, 8), jnp.float32)
    zeros16 = jnp.zeros((NACC, 16), jnp.float32)

    xpad = jnp.pad(x, ((0, 0), (0, 8 - IN_DIM))).at[:, 4].set(1.0)

    P = _sc_agg(jnp.broadcast_to(xpad.reshape(1, N, 8), (NC, N, 8)),
                src_p.reshape(-1, L1_LANES), dst_p.reshape(-1, L1_LANES),
                zeros8, split_edges=True, lanes=L1_LANES, sub=L1_SUB, width=8)

    W1p = jnp.pad(W1, ((0, 8 - IN_DIM), (0, 0)))
    h1t, dinv8 = _tc_layer1(P, W1p, b1.reshape(1, HID))

    A = _sc_agg(h1t,
                src_p.reshape(-1, L2_LANES), dst_p.reshape(-1, L2_LANES),
                zeros16, split_edges=False, lanes=L2_LANES, sub=L2_SUB, width=16)

    W2s = W2.reshape(NC, 16, HID)
    return _tc_layer2(A, dinv8, W2s, b2.reshape(1, HID))


# speedup vs baseline: 1.2240x; 1.2240x over previous
import functools
import jax
import jax.numpy as jnp
from jax import lax
from jax.experimental import pallas as pl
from jax.experimental.pallas import tpu as pltpu
from jax.experimental.pallas import tpu_sc as plsc

N = 100000
E = 6400000
IN_DIM = 4
HID = 32

NC = 2
NS = 16

L1_LANES, L1_SUB = 256, 3
L2_LANES, L2_SUB = 256, 3
C1 = L1_LANES * L1_SUB
C2 = L2_LANES * L2_SUB

import math
_GRAIN = math.lcm(NC * NS * C1, NS * C2)
EP = (E + _GRAIN - 1) // _GRAIN * _GRAIN
EPX = EP + max(C1, C2)
NACC = ((N + 1) + NS * 8 - 1) // (NS * 8) * (NS * 8)
ZCH = NACC // NS

_MESH = plsc.VectorSubcoreMesh(core_axis_name="c", subcore_axis_name="s")


def _sc_agg(table, idx2d, dst2d, zeros, split_edges, lanes, sub, width):
    chunk = lanes * sub
    idxrows = EP // lanes
    rows_per_sub = idxrows // ((NC * NS) if split_edges else NS)
    n_bodies = rows_per_sub // sub
    stacked = table.shape[0] == NC

    @functools.partial(
        pl.kernel,
        out_type=jax.ShapeDtypeStruct((NC, NACC, width), jnp.float32),
        mesh=_MESH,
        scratch_types=[
            pltpu.VMEM((2, sub, lanes), jnp.int32),
            pltpu.VMEM((2, sub, lanes), jnp.int32),
            pltpu.VMEM((2 * chunk, width), jnp.float32),
            pltpu.VMEM_SHARED((NACC, width), jnp.float32),
            pltpu.SemaphoreType.DMA,
            pltpu.SemaphoreType.DMA,
            pltpu.SemaphoreType.DMA,
            pltpu.SemaphoreType.DMA,
        ],
        compiler_params=pltpu.CompilerParams(use_tc_tiling_on_sc=False),
    )
    def k(table_h, idx_h, dst_h, zeros_h, out_h,
          sbuf, dbuf, rows_v, acc, gsem, ssem, isem, dsem):
        c = lax.axis_index("c")
        s = lax.axis_index("s")
        tab = table_h.at[c] if stacked else table_h.at[0]
        zb = s * ZCH
        pltpu.sync_copy(zeros_h.at[pl.ds(zb, ZCH)], acc.at[pl.ds(zb, ZCH)])
        plsc.subcore_barrier()

        if split_edges:
            row0 = (c * NS + s) * rows_per_sub
        else:
            row0 = s * rows_per_sub

        def drain_scatters(sem):
            for j in range(sub):
                pltpu.make_async_copy(
                    zeros_h.at[pl.ds(0, lanes)],
                    rows_v.at[pl.ds(j * lanes, lanes)], sem).wait()

        def drain_ibuf(buf, sem):
            pltpu.make_async_copy(idx_h.at[pl.ds(row0, sub)], buf, sem).wait()

        pltpu.sync_copy(idx_h.at[pl.ds(row0, sub)], sbuf.at[0])

        def body(i, carry):
            p = i & 1

            @pl.when(i > 1)
            def _():
                drain_scatters(ssem)

            pltpu.async_copy(dst_h.at[pl.ds(row0 + i * sub, sub)],
                             dbuf.at[p], dsem)

            @pl.when(i > 0)
            def _():
                drain_ibuf(sbuf.at[p], isem)

            g = [
                pltpu.async_copy(tab.at[sbuf.at[p, j]],
                                 rows_v.at[pl.ds(p * chunk + j * lanes, lanes)],
                                 gsem)
                for j in range(sub)
            ]
            pltpu.async_copy(idx_h.at[pl.ds(row0 + (i + 1) * sub, sub)],
                             sbuf.at[1 - p], isem)
            for cp in g:
                cp.wait()
            drain_ibuf(dbuf.at[p], dsem)
            for j in range(sub):
                pltpu.async_copy(rows_v.at[pl.ds(p * chunk + j * lanes, lanes)],
                                 acc.at[dbuf.at[p, j]], ssem, add=True)
            return carry

        lax.fori_loop(0, n_bodies, body, 0)
        drain_scatters(ssem)
        drain_scatters(ssem)
        drain_ibuf(sbuf.at[0], isem)
        plsc.subcore_barrier()
        pltpu.sync_copy(acc.at[pl.ds(zb, ZCH)], out_h.at[c, pl.ds(zb, ZCH)])

    return k(table, idx2d, dst2d, zeros)


BR = 1000


def _tc_layer1(P, W1p, b1):
    def body(p_ref, w_ref, b_ref, h_ref, dinv_ref):
        sblk = p_ref[0] + p_ref[1]
        dinv = 1.0 / jnp.maximum(sblk[:, 4:5], 1.0)
        mean = sblk * dinv
        h = jnp.dot(mean, w_ref[...], preferred_element_type=jnp.float32)
        h = jnp.maximum(h + b_ref[...], 0.0)
        h_ref[0] = h[:, :16]
        h_ref[1] = h[:, 16:]
        dinv_ref[...] = jnp.broadcast_to(dinv, (BR, 8))

    return pl.pallas_call(
        body,
        grid=(N // BR,),
        in_specs=[
            pl.BlockSpec((NC, BR, 16), lambda i: (0, i, 0)),
            pl.BlockSpec((16, HID), lambda i: (0, 0)),
            pl.BlockSpec((1, HID), lambda i: (0, 0)),
        ],
        out_specs=[
            pl.BlockSpec((NC, BR, 16), lambda i: (0, i, 0)),
            pl.BlockSpec((BR, 8), lambda i: (i, 0)),
        ],
        out_shape=[
            jax.ShapeDtypeStruct((NC, N, 16), jnp.float32),
            jax.ShapeDtypeStruct((N, 8), jnp.float32),
        ],
    )(P, W1p, b1)


def _tc_layer2(A, dinv8, W2s, b2):
    def body(a_ref, d_ref, w_ref, b_ref, o_ref):
        acc = jnp.dot(a_ref[0], w_ref[0], preferred_element_type=jnp.float32)
        acc += jnp.dot(a_ref[1], w_ref[1], preferred_element_type=jnp.float32)
        o_ref[...] = jnp.maximum(acc * d_ref[:, 0:1] + b_ref[...], 0.0)

    return pl.pallas_call(
        body,
        grid=(N // BR,),
        in_specs=[
            pl.BlockSpec((NC, BR, 16), lambda i: (0, i, 0)),
            pl.BlockSpec((BR, 8), lambda i: (i, 0)),
            pl.BlockSpec((NC, 16, HID), lambda i: (0, 0, 0)),
            pl.BlockSpec((1, HID), lambda i: (0, 0)),
        ],
        out_specs=pl.BlockSpec((BR, HID), lambda i: (i, 0)),
        out_shape=jax.ShapeDtypeStruct((N, HID), jnp.float32),
    )(A, dinv8, W2s, b2)


@jax.jit
def kernel(x, edge_index, W1, b1, W2, b2):
    src = edge_index[0]
    dst = edge_index[1]

    pad = EPX - E
    src_p = jnp.concatenate([src, jnp.zeros((pad,), jnp.int32)])
    dst_p = jnp.concatenate([dst, jnp.full((pad,), N, jnp.int32)])

    zeros16 = jnp.zeros((NACC, 16), jnp.float32)

    xpad = jnp.pad(x, ((0, 0), (0, 16 - IN_DIM))).at[:, 4].set(1.0)

    P = _sc_agg(xpad.reshape(1, N, 16),
                src_p.reshape(-1, L1_LANES), dst_p.reshape(-1, L1_LANES),
                zeros16, split_edges=True, lanes=L1_LANES, sub=L1_SUB, width=16)

    W1p = jnp.pad(W1, ((0, 16 - IN_DIM), (0, 0)))
    h1t, dinv8 = _tc_layer1(P, W1p, b1.reshape(1, HID))

    A = _sc_agg(h1t,
                src_p.reshape(-1, L2_LANES), dst_p.reshape(-1, L2_LANES),
                zeros16, split_edges=False, lanes=L2_LANES, sub=L2_SUB, width=16)

    W2s = W2.reshape(NC, 16, HID)
    return _tc_layer2(A, dinv8, W2s, b2.reshape(1, HID))
